# baseline (speedup 1.0000x reference)
import functools

import jax
import jax.numpy as jnp
from jax import lax
from jax.experimental import pallas as pl
from jax.experimental.pallas import tpu as pltpu
from jax.experimental.pallas import tpu_sc as plsc

D = 128
L = 16
KD = D // L
E = 2
S = 16
NB = 8


def _body(self_hbm, ent_hbm, edge_hbm, mask_hbm, sv_hbm, nv_hbm,
          edge_v, mask_v, ent_v, self_v, nv_v, sv_v):
    nc = 2
    wid = lax.axis_index("s") * nc + lax.axis_index("c")
    n_total = self_hbm.shape[0]
    nodes_per_w = n_total // 32
    num_chunks = nodes_per_w // NB
    base = wid * nodes_per_w

    def chunk_body(ci, carry):
        n0 = base + ci * NB
        pltpu.sync_copy(edge_hbm.at[pl.ds(n0, NB)], edge_v)
        pltpu.sync_copy(mask_hbm.at[pl.ds(n0, NB)], mask_v)
        pltpu.sync_copy(ent_hbm.at[pl.ds(n0, NB)], ent_v)
        pltpu.sync_copy(self_hbm.at[pl.ds(n0, NB)], self_v)

        def node_body(n, c2):
            sv_acc = [self_v[n, pl.ds(k * L, L)] for k in range(KD)]
            for e in range(E):
                m_vec = mask_v[n, e, :]
                cnt_vec = jnp.zeros((L,), jnp.float32)
                acc = [jnp.zeros((L,), jnp.float32) for _ in range(KD)]
                for s in range(S):
                    ms = jnp.broadcast_to(m_vec[s], (L,))
                    cnt_vec = cnt_vec + ms
                    for k in range(KD):
                        acc[k] = acc[k] + ms * edge_v[n, e, s, pl.ds(k * L, L)]
                denom = jnp.where(cnt_vec == 0.0,
                                  jnp.ones((L,), jnp.float32), cnt_vec)
                for k in range(KD):
                    nv_k = ent_v[n, e, pl.ds(k * L, L)] + acc[k] / denom
                    nv_v[n, e, pl.ds(k * L, L)] = nv_k
                    sv_acc[k] = sv_acc[k] + 0.5 * nv_k
            for k in range(KD):
                sv_v[n, pl.ds(k * L, L)] = sv_acc[k]
            return c2

        lax.fori_loop(0, NB, node_body, 0, unroll=False)
        pltpu.sync_copy(nv_v, nv_hbm.at[pl.ds(n0, NB)])
        pltpu.sync_copy(sv_v, sv_hbm.at[pl.ds(n0, NB)])
        return carry

    lax.fori_loop(0, num_chunks, chunk_body, 0, unroll=False)


@jax.jit
def kernel(self_vectors, neighbor_entity_vectors, neighbor_edge_vectors,
           masks):
    bs, P, d = self_vectors.shape
    n = bs * P
    self2 = self_vectors.reshape(n, d)
    ent2 = neighbor_entity_vectors.reshape(n, E, d)
    edge2 = neighbor_edge_vectors.reshape(n, E, S, d)
    mask2 = masks.reshape(n, E, S)

    mesh = plsc.VectorSubcoreMesh(core_axis_name="c", subcore_axis_name="s")
    sv, nv = pl.kernel(
        _body,
        out_type=(
            jax.ShapeDtypeStruct((n, d), jnp.float32),
            jax.ShapeDtypeStruct((n, E, d), jnp.float32),
        ),
        mesh=mesh,
        scratch_types=[
            pltpu.VMEM((NB, E, S, d), jnp.float32),
            pltpu.VMEM((NB, E, S), jnp.float32),
            pltpu.VMEM((NB, E, d), jnp.float32),
            pltpu.VMEM((NB, d), jnp.float32),
            pltpu.VMEM((NB, E, d), jnp.float32),
            pltpu.VMEM((NB, d), jnp.float32),
        ],
    )(self2, ent2, edge2, mask2)
    return sv.reshape(bs, P, d), nv.reshape(bs, P, E, d)


# speedup vs baseline: 2.2864x; 2.2864x over previous
import functools

import jax
import jax.numpy as jnp
from jax import lax
from jax.experimental import pallas as pl
from jax.experimental.pallas import tpu as pltpu
from jax.experimental.pallas import tpu_sc as plsc

D = 128
L = 16
KD = D // L
E = 2
S = 16
NB = 8
NBUF = 2


def _body(self_hbm, ent_hbm, edge_hbm, mask_hbm, sv_hbm, nv_hbm,
          edge_v, mask_v, ent_v, self_v, nv_v, sv_v, sem0, sem1):
    nc = 2
    wid = lax.axis_index("s") * nc + lax.axis_index("c")
    n_total = self_hbm.shape[0]
    nodes_per_w = n_total // 32
    num_chunks = nodes_per_w // NB
    base = wid * nodes_per_w
    sems = (sem0, sem1)

    def in_copies(ci, b):
        n0 = base + ci * NB
        return (
            pltpu.make_async_copy(edge_hbm.at[pl.ds(n0, NB)], edge_v.at[b],
                                  sems[b]),
            pltpu.make_async_copy(mask_hbm.at[pl.ds(n0, NB)], mask_v.at[b],
                                  sems[b]),
            pltpu.make_async_copy(ent_hbm.at[pl.ds(n0, NB)], ent_v.at[b],
                                  sems[b]),
            pltpu.make_async_copy(self_hbm.at[pl.ds(n0, NB)], self_v.at[b],
                                  sems[b]),
        )

    def start_in(ci, b):
        for c in in_copies(ci, b):
            c.start()

    def wait_in(ci, b):
        for c in in_copies(ci, b):
            c.wait()

    start_in(0, 0)
    start_in(1, 1)

    def pair_body(ci2, carry):
        for b in range(NBUF):
            ci = ci2 * NBUF + b
            n0 = base + ci * NB
            wait_in(ci, b)

            def node_body(n, c2):
                sv_acc = [self_v[b, n, pl.ds(k * L, L)] for k in range(KD)]
                for e in range(E):
                    m_vec = mask_v[b, n, e, :]
                    cnt_vec = jnp.zeros((L,), jnp.float32)
                    acc = [jnp.zeros((L,), jnp.float32) for _ in range(KD)]
                    for s in range(S):
                        ms = jnp.broadcast_to(m_vec[s], (L,))
                        cnt_vec = cnt_vec + ms
                        for k in range(KD):
                            acc[k] = acc[k] + ms * edge_v[b, n, e, s,
                                                          pl.ds(k * L, L)]
                    denom = jnp.where(cnt_vec == 0.0,
                                      jnp.ones((L,), jnp.float32), cnt_vec)
                    for k in range(KD):
                        nv_k = ent_v[b, n, e, pl.ds(k * L, L)] + acc[k] / denom
                        nv_v[b, n, e, pl.ds(k * L, L)] = nv_k
                        sv_acc[k] = sv_acc[k] + 0.5 * nv_k
                for k in range(KD):
                    sv_v[b, n, pl.ds(k * L, L)] = sv_acc[k]
                return c2

            lax.fori_loop(0, NB, node_body, 0, unroll=False)
            pltpu.sync_copy(nv_v.at[b], nv_hbm.at[pl.ds(n0, NB)])
            pltpu.sync_copy(sv_v.at[b], sv_hbm.at[pl.ds(n0, NB)])

            @pl.when(ci + NBUF < num_chunks)
            def _():
                start_in(ci + NBUF, b)
        return carry

    lax.fori_loop(0, num_chunks // NBUF, pair_body, 0, unroll=False)


@jax.jit
def kernel(self_vectors, neighbor_entity_vectors, neighbor_edge_vectors,
           masks):
    bs, P, d = self_vectors.shape
    n = bs * P
    self2 = self_vectors.reshape(n, d)
    ent2 = neighbor_entity_vectors.reshape(n, E, d)
    edge2 = neighbor_edge_vectors.reshape(n, E, S, d)
    mask2 = masks.reshape(n, E, S)

    mesh = plsc.VectorSubcoreMesh(core_axis_name="c", subcore_axis_name="s")
    sv, nv = pl.kernel(
        _body,
        out_type=(
            jax.ShapeDtypeStruct((n, d), jnp.float32),
            jax.ShapeDtypeStruct((n, E, d), jnp.float32),
        ),
        mesh=mesh,
        scratch_types=[
            pltpu.VMEM((NBUF, NB, E, S, d), jnp.float32),
            pltpu.VMEM((NBUF, NB, E, S), jnp.float32),
            pltpu.VMEM((NBUF, NB, E, d), jnp.float32),
            pltpu.VMEM((NBUF, NB, d), jnp.float32),
            pltpu.VMEM((NBUF, NB, E, d), jnp.float32),
            pltpu.VMEM((NBUF, NB, d), jnp.float32),
            pltpu.SemaphoreType.DMA,
            pltpu.SemaphoreType.DMA,
        ],
    )(self2, ent2, edge2, mask2)
    return sv.reshape(bs, P, d), nv.reshape(bs, P, E, d)
